# baseline (speedup 1.0000x reference)
import jax
import jax.numpy as jnp
from jax.experimental import pallas as pl

NUM_PATCHES = 64
PATCH_AREA = 108
PROJ_DIM = 128

BLOCK_BATCH = 256


def _patch_encoder_kernel(x_ref, w_ref, pb_ref, o_ref):
    x = x_ref[...].reshape(-1, PATCH_AREA)
    y = jax.lax.dot_general(
        x, w_ref[...],
        dimension_numbers=(((1,), (0,)), ((), ())),
        preferred_element_type=jnp.float32,
    )
    o_ref[...] = y.reshape(-1, NUM_PATCHES, PROJ_DIM) + pb_ref[...]


@jax.jit
def kernel(patches, W, b, pos_table):
    batch = patches.shape[0]
    pb = (pos_table + b[None, :])[None]
    grid = (batch // BLOCK_BATCH,)
    return pl.pallas_call(
        _patch_encoder_kernel,
        grid=grid,
        in_specs=[
            pl.BlockSpec((BLOCK_BATCH, NUM_PATCHES, PATCH_AREA), lambda i: (i, 0, 0)),
            pl.BlockSpec((PATCH_AREA, PROJ_DIM), lambda i: (0, 0)),
            pl.BlockSpec((1, NUM_PATCHES, PROJ_DIM), lambda i: (0, 0, 0)),
        ],
        out_specs=pl.BlockSpec((BLOCK_BATCH, NUM_PATCHES, PROJ_DIM), lambda i: (i, 0, 0)),
        out_shape=jax.ShapeDtypeStruct((batch, NUM_PATCHES, PROJ_DIM), jnp.float32),
    )(patches, W, pb)
